# baseline (speedup 1.0000x reference)
import functools

import jax
import jax.numpy as jnp
from jax import lax
from jax.experimental import pallas as pl
from jax.experimental.pallas import tpu as pltpu
from jax.experimental.pallas import tpu_sc as plsc

_B = 32
_N = 2048
_C = 1000
_M = 128

_NC = 2
_NS = 16
_LANES = 16
_ROWW = 1008
_NVREG = _ROWW // _LANES
_CH = 32
_NCHUNK = _M // _CH
_PAD = -3.0e38


def _sc_body(class_hbm, labels_hbm, sumexp_hbm, picked_hbm,
             buf0, buf1, labels_v, stage_sum2d, stage_pick,
             sem0, sem1, sem_l):
    wid = lax.axis_index("s") * _NC + lax.axis_index("c")

    lcp = pltpu.async_copy(labels_hbm.at[wid], labels_v, sem_l)

    padv = jnp.full((_LANES,), _PAD, jnp.float32)
    for buf in (buf0, buf1):
        for r in range(_CH):
            buf[pl.ds(r * _ROWW + (_ROWW - _LANES), _LANES)] = padv

    bufs = (buf0, buf1)
    sems = (sem0, sem1)

    def issue(chunk):
        buf, sem = bufs[chunk % 2], sems[chunk % 2]
        return [
            pltpu.async_copy(
                class_hbm.at[wid, chunk * _CH + r, :],
                buf.at[pl.ds(r * _ROWW, _C)],
                sem,
            )
            for r in range(_CH)
        ]

    pending = {0: issue(0)}
    lcp.wait()
    il = lax.iota(jnp.int32, _LANES)

    for chunk in range(_NCHUNK):
        if chunk + 1 < _NCHUNK:
            pending[chunk + 1] = issue(chunk + 1)
        for d in pending.pop(chunk):
            d.wait()
        buf = bufs[chunk % 2]
        for g in range(_CH // _LANES):
            goff = (chunk * (_CH // _LANES) + g) * _LANES

            def row_body(rl, carry, _g=g, _goff=goff):
                base = (_g * _LANES + rl) * _ROWW
                sv = jnp.zeros((_LANES,), jnp.float32)
                for j in range(_NVREG):
                    sv = sv + jnp.exp(buf[pl.ds(base + j * _LANES, _LANES)])
                plsc.store_scatter(
                    stage_sum2d, [il, jnp.full((_LANES,), _goff + rl,
                                               jnp.int32)], sv)
                return carry

            lax.fori_loop(0, _LANES, row_body, 0)
            labels16 = labels_v[pl.ds(goff, _LANES)]
            idx = (g * _LANES + il) * _ROWW + labels16
            pickvec = plsc.load_gather(buf, [idx])
            stage_pick[pl.ds(goff, _LANES)] = pickvec

    pltpu.sync_copy(stage_sum2d, sumexp_hbm.at[wid])
    pltpu.sync_copy(stage_pick, picked_hbm.at[wid])


_sc_call = functools.partial(
    pl.kernel,
    out_type=[
        jax.ShapeDtypeStruct((_B, _LANES, _M), jnp.float32),
        jax.ShapeDtypeStruct((_B, _M), jnp.float32),
    ],
    mesh=plsc.VectorSubcoreMesh(
        core_axis_name="c", subcore_axis_name="s",
        num_cores=_NC, num_subcores=_NS),
    compiler_params=pltpu.CompilerParams(
        needs_layout_passes=False, use_tc_tiling_on_sc=False),
    scratch_types=[
        pltpu.VMEM((_CH * _ROWW,), jnp.float32),
        pltpu.VMEM((_CH * _ROWW,), jnp.float32),
        pltpu.VMEM((_M,), jnp.int32),
        pltpu.VMEM((_LANES, _M), jnp.float32),
        pltpu.VMEM((_M,), jnp.float32),
        pltpu.SemaphoreType.DMA,
        pltpu.SemaphoreType.DMA,
        pltpu.SemaphoreType.DMA,
    ],
)(_sc_body)


def _tc_body(obj_ref, pbox_ref, tbox_ref, sumexp_ref, picked_ref, out_ref):
    x = obj_ref[...]
    col = lax.broadcasted_iota(jnp.int32, (_B, _N), 1)
    t = (col < _M).astype(jnp.float32)
    bce = jnp.maximum(x, 0.0) - x * t + jnp.log1p(jnp.exp(-jnp.abs(x)))
    obj_loss = jnp.sum(bce) * (1.0 / (_B * _N))

    box_loss = jnp.sum(jnp.abs(pbox_ref[...] - tbox_ref[...])) * (
        1.0 / (_B * _M * 4))

    lse = jnp.log(jnp.sum(sumexp_ref[...], axis=1))
    class_loss = jnp.sum(lse - picked_ref[...]) * (1.0 / (_B * _M))

    out_ref[0] = box_loss + obj_loss + class_loss
    out_ref[1] = box_loss
    out_ref[2] = obj_loss
    out_ref[3] = class_loss


def kernel(pred_boxes, pred_obj, pred_class, tgt_boxes, tgt_labels):
    labels = tgt_labels.astype(jnp.int32)
    sumexp, picked = _sc_call(pred_class, labels)

    pb = pred_boxes[:, :_M, :].reshape(_B, _M * 4)
    tb = tgt_boxes.reshape(_B, _M * 4)

    out = pl.pallas_call(
        _tc_body,
        out_shape=jax.ShapeDtypeStruct((4,), jnp.float32),
        out_specs=pl.BlockSpec(memory_space=pltpu.SMEM),
    )(pred_obj, pb, tb, sumexp, picked)
    return (out[0], out[1], out[2], out[3])


# speedup vs baseline: 6.3495x; 6.3495x over previous
import functools

import jax
import jax.numpy as jnp
from jax import lax
from jax.experimental import pallas as pl
from jax.experimental.pallas import tpu as pltpu
from jax.experimental.pallas import tpu_sc as plsc

_B = 32
_N = 2048
_C = 1000
_M = 128

_NC = 2
_NS = 16
_LANES = 16
_NFULL = _C // _LANES
_TAIL = _C - _NFULL * _LANES
_CH = 32
_NCHUNK = _M // _CH
_SROW = _M + 1


def _sc_body(cls_hbm, labels_hbm, sumexp_hbm, picked_hbm,
             buf0, buf1, labels_v, stage_sum2d, stage_pick,
             sem0, sem1, sem_l):
    wid = lax.axis_index("s") * _NC + lax.axis_index("c")

    lcp = pltpu.async_copy(labels_hbm.at[wid], labels_v, sem_l)

    bufs = (buf0, buf1)
    sems = (sem0, sem1)

    def issue(chunk):
        return pltpu.async_copy(
            cls_hbm.at[wid, pl.ds(chunk * _CH, _CH), :],
            bufs[chunk % 2], sems[chunk % 2])

    pending = {0: issue(0)}
    lcp.wait()
    il = lax.iota(jnp.int32, _LANES)
    zero = jnp.zeros((_LANES,), jnp.float32)

    for chunk in range(_NCHUNK):
        if chunk + 1 < _NCHUNK:
            pending[chunk + 1] = issue(chunk + 1)
        pending.pop(chunk).wait()
        buf = bufs[chunk % 2]
        for g in range(_CH // _LANES):
            goff = chunk * _CH + g * _LANES

            def row_body(rl, carry, _g=g, _goff=goff):
                r = _g * _LANES + rl
                acc = [zero, zero, zero, zero]
                for j in range(_NFULL):
                    acc[j % 4] = acc[j % 4] + jnp.exp(
                        buf[r, pl.ds(j * _LANES, _LANES)])
                t = jnp.exp(buf[r, pl.ds(_C - _LANES, _LANES)])
                acc[0] = acc[0] + jnp.where(il >= _LANES - _TAIL, t, 0.0)
                sv = (acc[0] + acc[1]) + (acc[2] + acc[3])
                plsc.store_scatter(
                    stage_sum2d,
                    [il, jnp.full((_LANES,), _goff + rl, jnp.int32)], sv)
                return carry

            lax.fori_loop(0, _LANES, row_body, 0)
            labels16 = labels_v[pl.ds(goff, _LANES)]
            pickvec = plsc.load_gather(buf, [g * _LANES + il, labels16])
            stage_pick[pl.ds(goff, _LANES)] = pickvec

    pltpu.sync_copy(stage_sum2d, sumexp_hbm.at[wid])
    pltpu.sync_copy(stage_pick, picked_hbm.at[wid])


_sc_call = functools.partial(
    pl.kernel,
    out_type=[
        jax.ShapeDtypeStruct((_B, _LANES, _SROW), jnp.float32),
        jax.ShapeDtypeStruct((_B, _M), jnp.float32),
    ],
    mesh=plsc.VectorSubcoreMesh(
        core_axis_name="c", subcore_axis_name="s",
        num_cores=_NC, num_subcores=_NS),
    compiler_params=pltpu.CompilerParams(
        needs_layout_passes=False, use_tc_tiling_on_sc=False),
    scratch_types=[
        pltpu.VMEM((_CH, _C), jnp.float32),
        pltpu.VMEM((_CH, _C), jnp.float32),
        pltpu.VMEM((_M,), jnp.int32),
        pltpu.VMEM((_LANES, _SROW), jnp.float32),
        pltpu.VMEM((_M,), jnp.float32),
        pltpu.SemaphoreType.DMA,
        pltpu.SemaphoreType.DMA,
        pltpu.SemaphoreType.DMA,
    ],
)(_sc_body)


def _tc_body(obj_ref, pbox_ref, tbox_ref, sumexp_ref, picked_ref, out_ref):
    x = obj_ref[...]
    col = lax.broadcasted_iota(jnp.int32, (_B, _N), 1)
    t = (col < _M).astype(jnp.float32)
    bce = jnp.maximum(x, 0.0) - x * t + jnp.log1p(jnp.exp(-jnp.abs(x)))
    obj_loss = jnp.sum(bce) * (1.0 / (_B * _N))

    box_loss = jnp.sum(jnp.abs(pbox_ref[...] - tbox_ref[...])) * (
        1.0 / (_B * _M * 4))

    sums = jnp.sum(sumexp_ref[...], axis=1)[:, :_M]
    lse = jnp.log(sums)
    class_loss = jnp.sum(lse - picked_ref[...]) * (1.0 / (_B * _M))

    out_ref[0] = box_loss + obj_loss + class_loss
    out_ref[1] = box_loss
    out_ref[2] = obj_loss
    out_ref[3] = class_loss


def kernel(pred_boxes, pred_obj, pred_class, tgt_boxes, tgt_labels):
    labels = tgt_labels.astype(jnp.int32)
    cls = pred_class[:, :_M, :]
    sumexp, picked = _sc_call(cls, labels)

    pb = pred_boxes[:, :_M, :].reshape(_B, _M * 4)
    tb = tgt_boxes.reshape(_B, _M * 4)

    out = pl.pallas_call(
        _tc_body,
        out_shape=jax.ShapeDtypeStruct((4,), jnp.float32),
        out_specs=pl.BlockSpec(memory_space=pltpu.SMEM),
    )(pred_obj, pb, tb, sumexp, picked)
    return (out[0], out[1], out[2], out[3])
